# Initial kernel scaffold; baseline (speedup 1.0000x reference)
import jax
import jax.numpy as jnp
from jax.experimental import pallas as pl


def kernel(x, edge_index_0, edge_index_1, edge_index_2, Wl, bl, Wr, att, Wc, bc):
    raise NotImplementedError("write your pallas kernel here")


# speedup vs baseline: 1.1584x; 1.1584x over previous
import jax
import jax.numpy as jnp
from jax import lax
from jax.experimental import pallas as pl
from jax.experimental.pallas import tpu as pltpu

N = 100000
E = 500000
D = 128
T = 3
BLK = 2000


def _tc_combine(x, summed, cnt, wlt, bl, wrt, att, wc3, bc):
    grid = (N // BLK,)

    def body(x_ref, s_ref, c_ref, wl_ref, bl_ref, wr_ref, att_ref, wc_ref,
             bc_ref, o_ref):
        xb = x_ref[...]
        y = jnp.broadcast_to(bc_ref[...], (BLK, D)).astype(jnp.float32)
        for t in range(T):
            agg = s_ref[t] / jnp.maximum(c_ref[t][:, 0:1], 1.0)
            h = (jnp.dot(agg, wl_ref[t], preferred_element_type=jnp.float32)
                 + bl_ref[t][None, :]
                 + jnp.dot(xb, wr_ref[t], preferred_element_type=jnp.float32))
            s2 = jnp.sum(h * h, axis=-1, keepdims=True)
            scale = att_ref[t] * lax.rsqrt(jnp.maximum(s2, 1e-24))
            y = y + jnp.dot(h * scale, wc_ref[t],
                            preferred_element_type=jnp.float32)
        o_ref[...] = y

    return pl.pallas_call(
        body,
        grid=grid,
        in_specs=[
            pl.BlockSpec((BLK, D), lambda i: (i, 0)),
            pl.BlockSpec((T, BLK, D), lambda i: (0, i, 0)),
            pl.BlockSpec((T, BLK, 16), lambda i: (0, i, 0)),
            pl.BlockSpec((T, D, D), lambda i: (0, 0, 0)),
            pl.BlockSpec((T, D), lambda i: (0, 0)),
            pl.BlockSpec((T, D, D), lambda i: (0, 0, 0)),
            pl.BlockSpec(memory_space=pltpu.SMEM),
            pl.BlockSpec((T, D, D), lambda i: (0, 0, 0)),
            pl.BlockSpec((1, D), lambda i: (0, 0)),
        ],
        out_specs=pl.BlockSpec((BLK, D), lambda i: (i, 0)),
        out_shape=jax.ShapeDtypeStruct((N, D), jnp.float32),
    )(x, summed, cnt, wlt, bl, wrt, att, wc3, bc)


def kernel(x, edge_index_0, edge_index_1, edge_index_2, Wl, bl, Wr, att, Wc,
           bc):
    sums = []
    cnts = []
    for ei in (edge_index_0, edge_index_1, edge_index_2):
        src, dst = ei[0], ei[1]
        sums.append(jnp.zeros((N, D), jnp.float32).at[dst].add(x[src]))
        cnts.append(jnp.zeros((N,), jnp.float32).at[dst].add(1.0))
    summed = jnp.stack(sums)
    cnt = jnp.broadcast_to(jnp.stack(cnts)[:, :, None], (T, N, 16))

    wlt = Wl.transpose(0, 2, 1)
    wrt = Wr.transpose(0, 2, 1)
    wc3 = Wc.reshape(D, T, D).transpose(1, 2, 0)
    return _tc_combine(x, summed, cnt, wlt, bl, wrt, att, wc3,
                       bc.reshape(1, D))
